# baseline (speedup 1.0000x reference)
import functools
import math

import jax
import jax.numpy as jnp
from jax import lax
from jax.experimental import pallas as pl
from jax.experimental.pallas import tpu as pltpu
from jax.experimental.pallas import tpu_sc as plsc

_VOCAB = 1000000
_T = 200
_D = 64
_B = 4096

_NUM_CORES = 2
_NUM_SUBCORES = 16
_NW = _NUM_CORES * _NUM_SUBCORES
_SEQ_PER_W = _B // _NW

_SPLIT = 104

_SCALE = math.sqrt(_D)


def _make_pe():
    pos = jnp.arange(_T, dtype=jnp.float32)[:, None]
    i = jnp.arange(0, _D, 2, dtype=jnp.float32)[None, :]
    angle = pos / jnp.power(10000.0, 2.0 * i / _D)
    pe = jnp.zeros((_T, _D), dtype=jnp.float32)
    pe = pe.at[:, 0::2].set(jnp.sin(angle))
    pe = pe.at[:, 1::2].set(jnp.cos(angle))
    return pe


def _sc_body(idx_hbm, table_hbm, pe_hbm, out_hbm, idx_v, rows_v, pe_v,
             gsem, osem):
    wid = lax.axis_index("s") * _NUM_CORES + lax.axis_index("c")
    pltpu.sync_copy(pe_hbm, pe_v)

    def seq_body(s, _):
        g = wid * _SEQ_PER_W + s
        pltpu.sync_copy(idx_hbm.at[pl.ds(g * _T, _T)], idx_v)
        c1 = pltpu.async_copy(
            table_hbm.at[idx_v.at[pl.ds(0, _SPLIT)]],
            rows_v.at[pl.ds(0, _SPLIT)], gsem)
        c2 = pltpu.async_copy(
            table_hbm.at[idx_v.at[pl.ds(_SPLIT, _T - _SPLIT)]],
            rows_v.at[pl.ds(_SPLIT, _T - _SPLIT)], gsem)
        c1.wait()
        c2.wait()

        def row_body(j, _):
            for k in range(_D // 16):
                sl = pl.ds(k * 16, 16)
                rows_v[j, sl] = rows_v[j, sl] * _SCALE + pe_v[j, sl]
            return 0

        lax.fori_loop(0, _T, row_body, 0, unroll=2)
        pltpu.async_copy(rows_v, out_hbm.at[pl.ds(g * _T, _T)], osem).wait()
        return 0

    lax.fori_loop(0, _SEQ_PER_W, seq_body, 0)


@jax.jit
def _run(idx_flat, table, pe):
    mesh = plsc.VectorSubcoreMesh(core_axis_name="c", subcore_axis_name="s")
    k = functools.partial(
        pl.kernel,
        mesh=mesh,
        out_type=jax.ShapeDtypeStruct((_B * _T, _D), jnp.float32),
        scratch_types=[
            pltpu.VMEM((_T,), jnp.int32),
            pltpu.VMEM((_T, _D), jnp.float32),
            pltpu.VMEM((_T, _D), jnp.float32),
            pltpu.SemaphoreType.DMA,
            pltpu.SemaphoreType.DMA,
        ],
        compiler_params=pltpu.CompilerParams(use_tc_tiling_on_sc=False),
    )(_sc_body)
    return k(idx_flat, table, pe)


def kernel(x, table):
    pe = _make_pe()
    idx_flat = x.reshape(-1).astype(jnp.int32)
    out = _run(idx_flat, table, pe)
    return out.reshape(_B, _T, _D)


# speedup vs baseline: 1.2810x; 1.2810x over previous
import functools
import math

import jax
import jax.numpy as jnp
from jax import lax
from jax.experimental import pallas as pl
from jax.experimental.pallas import tpu as pltpu
from jax.experimental.pallas import tpu_sc as plsc

_VOCAB = 1000000
_T = 200
_D = 64
_B = 4096

_NUM_CORES = 2
_NUM_SUBCORES = 16
_NW = _NUM_CORES * _NUM_SUBCORES
_SEQ_PER_W = _B // _NW
_IDX_PER_W = _SEQ_PER_W * _T

_SEQ_PER_CHUNK = 2
_CH = _SEQ_PER_CHUNK * _T
_NCH = _SEQ_PER_W // _SEQ_PER_CHUNK

_IDX_SPLITS = [(0, 128), (128, 128), (256, 128), (384, 16)]

_SCALE = math.sqrt(_D)


def _make_pe():
    pos = jnp.arange(_T, dtype=jnp.float32)[:, None]
    i = jnp.arange(0, _D, 2, dtype=jnp.float32)[None, :]
    angle = pos / jnp.power(10000.0, 2.0 * i / _D)
    pe = jnp.zeros((_T, _D), dtype=jnp.float32)
    pe = pe.at[:, 0::2].set(jnp.sin(angle))
    pe = pe.at[:, 1::2].set(jnp.cos(angle))
    return pe


def _sc_body(idx_hbm, table_hbm, pe_hbm, out_hbm, idx_v, rows_v, pe_v,
             gsem, osem):
    wid = lax.axis_index("s") * _NUM_CORES + lax.axis_index("c")

    pltpu.sync_copy(pe_hbm, pe_v)
    pltpu.sync_copy(idx_hbm.at[pl.ds(wid * _IDX_PER_W, _IDX_PER_W)], idx_v)

    def gather_descs(c, db):
        descs = []
        for off, n in _IDX_SPLITS:
            descs.append(pltpu.make_async_copy(
                table_hbm.at[idx_v.at[pl.ds(c * _CH + off, n)]],
                rows_v.at[db].at[pl.ds(off, n)],
                gsem))
        return descs

    def store_desc(c, db):
        return pltpu.make_async_copy(
            rows_v.at[db],
            out_hbm.at[pl.ds((wid * _NCH + c) * _CH, _CH)],
            osem)

    def compute(db):
        def row_body(j, _):
            for r0 in (0, _T):
                for k in range(_D // 16):
                    sl = pl.ds(k * 16, 16)
                    rows_v[db, r0 + j, sl] = (
                        rows_v[db, r0 + j, sl] * _SCALE + pe_v[j, sl])
            return 0

        lax.fori_loop(0, _T, row_body, 0, unroll=2)

    for d in gather_descs(0, 0):
        d.start()

    def outer(i, _):
        c0 = i * 2
        for db in range(2):
            c = c0 + db
            nb = 1 - db

            @pl.when(c >= 1)
            def _():
                store_desc(c - 1, nb).wait()

            @pl.when(c + 1 < _NCH)
            def _():
                for d in gather_descs(c + 1, nb):
                    d.start()

            for d in gather_descs(c, db):
                d.wait()
            compute(db)
            store_desc(c, db).start()
        return 0

    lax.fori_loop(0, _NCH // 2, outer, 0)
    store_desc(_NCH - 1, 1).wait()


@jax.jit
def _run(idx_flat, table, pe):
    mesh = plsc.VectorSubcoreMesh(core_axis_name="c", subcore_axis_name="s")
    k = functools.partial(
        pl.kernel,
        mesh=mesh,
        out_type=jax.ShapeDtypeStruct((_B * _T, _D), jnp.float32),
        scratch_types=[
            pltpu.VMEM((_IDX_PER_W,), jnp.int32),
            pltpu.VMEM((2, _CH, _D), jnp.float32),
            pltpu.VMEM((_T, _D), jnp.float32),
            pltpu.SemaphoreType.DMA,
            pltpu.SemaphoreType.DMA,
        ],
        compiler_params=pltpu.CompilerParams(use_tc_tiling_on_sc=False),
    )(_sc_body)
    return k(idx_flat, table, pe)


def kernel(x, table):
    pe = _make_pe()
    idx_flat = x.reshape(-1).astype(jnp.int32)
    out = _run(idx_flat, table, pe)
    return out.reshape(_B, _T, _D)


# speedup vs baseline: 2.3498x; 1.8343x over previous
import functools
import math

import jax
import jax.numpy as jnp
from jax import lax
from jax.experimental import pallas as pl
from jax.experimental.pallas import tpu as pltpu
from jax.experimental.pallas import tpu_sc as plsc

_VOCAB = 1000000
_T = 200
_D = 64
_B = 4096

_NUM_CORES = 2
_NUM_SUBCORES = 16
_D_PER_CORE = _D // _NUM_CORES
_BW = _B // _NUM_SUBCORES
_TB = 25
_NBLK = _T // _TB

_SCALE = math.sqrt(_D)


def _make_pe_splat():
    pos = jnp.arange(_T, dtype=jnp.float32)[:, None]
    i = jnp.arange(0, _D, 2, dtype=jnp.float32)[None, :]
    angle = pos / jnp.power(10000.0, 2.0 * i / _D)
    pe = jnp.zeros((_T, _D), dtype=jnp.float32)
    pe = pe.at[:, 0::2].set(jnp.sin(angle))
    pe = pe.at[:, 1::2].set(jnp.cos(angle))
    return jnp.repeat(pe.T[:, :, None], 16, axis=2).reshape(_D, _T * 16)


def _sc_body(xT, tabT, peS, outT, xv, dst, pe_v, ssem, psem, gsem, osem,
             xsem, spm):
    c = lax.axis_index("c")
    s = lax.axis_index("s")
    b0 = s * _BW
    dbase = c * _D_PER_CORE

    def tab_stage_desc(d):
        return pltpu.make_async_copy(tabT.at[d], spm, ssem)

    def pe_stage_desc(d):
        return pltpu.make_async_copy(peS.at[d], pe_v, psem)

    def gather_desc(t):
        return pltpu.make_async_copy(
            spm.at[xv.at[pl.ds(t * _BW, _BW)]],
            dst.at[pl.ds((t % _TB) * _BW, _BW)], gsem)

    def out_desc(t, d):
        return pltpu.make_async_copy(
            dst.at[pl.ds((t % _TB) * _BW, _BW)],
            outT.at[t, d, pl.ds(b0, _BW)], osem)

    def xdesc(t):
        return pltpu.make_async_copy(
            xT.at[t, pl.ds(b0, _BW)], xv.at[pl.ds(t * _BW, _BW)], xsem)

    def xfire(t, _):
        xdesc(t).start()
        return 0

    def xdrain(t, _):
        xdesc(t).wait()
        return 0

    lax.fori_loop(0, _T, xfire, 0)
    lax.fori_loop(0, _T, xdrain, 0)
    pe_stage_desc(dbase).start()

    @pl.when(s == 0)
    def _():
        tab_stage_desc(dbase).start()
        tab_stage_desc(dbase).wait()

    pe_stage_desc(dbase).wait()
    plsc.subcore_barrier()

    def dbody(i, _):
        d = dbase + i

        def block(blk, _):
            t0 = blk * _TB

            @pl.when(i + blk >= 1)
            def _():
                tprev = t0 - _TB
                dprev = jnp.where(blk >= 1, d, d - 1)

                def odrain(j, _):
                    out_desc((tprev + j) % _T, dprev).wait()
                    return 0

                lax.fori_loop(0, _TB, odrain, 0)

            def gfire(j, _):
                gather_desc(t0 + j).start()
                return 0

            lax.fori_loop(0, _TB, gfire, 0)

            def tbody(j, _):
                t = t0 + j
                gather_desc(t).wait()
                off = (t % _TB) * _BW
                pev = pe_v[pl.ds(t * 16, 16)]
                for k in range(_BW // 16):
                    sl = pl.ds(off + k * 16, 16)
                    dst[sl] = dst[sl] * _SCALE + pev
                out_desc(t, d).start()
                return 0

            lax.fori_loop(0, _TB, tbody, 0)
            return 0

        lax.fori_loop(0, _NBLK, block, 0)

        plsc.subcore_barrier()

        @pl.when(i + 1 < _D_PER_CORE)
        def _():
            @pl.when(s == 0)
            def _():
                tab_stage_desc(d + 1).start()
                tab_stage_desc(d + 1).wait()

            pe_stage_desc(d + 1).start()
            pe_stage_desc(d + 1).wait()

        plsc.subcore_barrier()
        return 0

    lax.fori_loop(0, _D_PER_CORE, dbody, 0)

    dlast = dbase + _D_PER_CORE - 1

    def odrain_last(j, _):
        out_desc(_T - _TB + j, dlast).wait()
        return 0

    lax.fori_loop(0, _TB, odrain_last, 0)


@jax.jit
def _run(xT, tabT, peS):
    mesh = plsc.VectorSubcoreMesh(core_axis_name="c", subcore_axis_name="s")
    k = functools.partial(
        pl.kernel,
        mesh=mesh,
        out_type=jax.ShapeDtypeStruct((_T, _D, _B), jnp.float32),
        scratch_types=[
            pltpu.VMEM((_T * _BW,), jnp.int32),
            pltpu.VMEM((_TB * _BW,), jnp.float32),
            pltpu.VMEM((_T * 16,), jnp.float32),
            pltpu.SemaphoreType.DMA,
            pltpu.SemaphoreType.DMA,
            pltpu.SemaphoreType.DMA,
            pltpu.SemaphoreType.DMA,
            pltpu.SemaphoreType.DMA,
            pltpu.VMEM_SHARED((_VOCAB,), jnp.float32),
        ],
    )(_sc_body)
    return k(xT, tabT, peS)


def kernel(x, table):
    peS = _make_pe_splat()
    outT = _run(x.T, table.T, peS)
    return outT.transpose(2, 0, 1)


# speedup vs baseline: 2.4331x; 1.0354x over previous
import functools
import math

import jax
import jax.numpy as jnp
from jax import lax
from jax.experimental import pallas as pl
from jax.experimental.pallas import tpu as pltpu
from jax.experimental.pallas import tpu_sc as plsc

_VOCAB = 1000000
_T = 200
_D = 64
_B = 4096

_NUM_CORES = 2
_NUM_SUBCORES = 16
_D_PER_CORE = _D // _NUM_CORES
_BW = _B // _NUM_SUBCORES
_TB = 25
_NBLK = _T // _TB

_SCALE = math.sqrt(_D)


def _make_pe_splat():
    pos = jnp.arange(_T, dtype=jnp.float32)[:, None]
    i = jnp.arange(0, _D, 2, dtype=jnp.float32)[None, :]
    angle = pos / jnp.power(10000.0, 2.0 * i / _D)
    pe = jnp.zeros((_T, _D), dtype=jnp.float32)
    pe = pe.at[:, 0::2].set(jnp.sin(angle))
    pe = pe.at[:, 1::2].set(jnp.cos(angle))
    return jnp.repeat(pe.T[:, :, None], 16, axis=2).reshape(_D, _T * 16)


def _sc_body(xT, tabT, peS, outT, xv, dst0, dst1, pe_v, ssem, psem, gsem,
             osem, xsem, spm):
    c = lax.axis_index("c")
    s = lax.axis_index("s")
    b0 = s * _BW
    dbase = c * _D_PER_CORE

    def tab_stage_desc(d):
        return pltpu.make_async_copy(tabT.at[d], spm, ssem)

    def pe_stage_desc(d):
        return pltpu.make_async_copy(peS.at[d], pe_v, psem)

    dsts = (dst0, dst1)

    def gather_desc(blk, p):
        return pltpu.make_async_copy(
            spm.at[xv.at[pl.ds(blk * _TB * _BW, _TB * _BW)]],
            dsts[p], gsem)

    def out_desc(t, d, p):
        return pltpu.make_async_copy(
            dsts[p].at[pl.ds((t % _TB) * _BW, _BW)],
            outT.at[t, d, pl.ds(b0, _BW)], osem)

    def xdesc(t):
        return pltpu.make_async_copy(
            xT.at[t, pl.ds(b0, _BW)], xv.at[pl.ds(t * _BW, _BW)], xsem)

    def xfire(t, _):
        xdesc(t).start()
        return 0

    def xdrain(t, _):
        xdesc(t).wait()
        return 0

    lax.fori_loop(0, _T, xfire, 0)
    lax.fori_loop(0, _T, xdrain, 0)
    pe_stage_desc(dbase).start()

    @pl.when(s == 0)
    def _():
        tab_stage_desc(dbase).start()
        tab_stage_desc(dbase).wait()

    pe_stage_desc(dbase).wait()
    plsc.subcore_barrier()

    def dbody(i, _):
        d = dbase + i

        def blockpair(bp, _):
            for p in range(2):
                blk = bp * 2 + p
                t0 = blk * _TB

                @pl.when(i * _NBLK + blk >= 2)
                def _(blk=blk, t0=t0, p=p):
                    tprev = (t0 - 2 * _TB) % _T
                    dprev = jnp.where(blk >= 2, d, d - 1)

                    def odrain(j, _):
                        out_desc(tprev + j, dprev, p).wait()
                        return 0

                    lax.fori_loop(0, _TB, odrain, 0)

                gather_desc(blk, p).start()

            for p in range(2):
                blk = bp * 2 + p
                t0 = blk * _TB
                gather_desc(blk, p).wait()

                def tbody(j, _, t0=t0, p=p):
                    t = t0 + j
                    off = j * _BW
                    pev = pe_v[pl.ds(t * 16, 16)]
                    for k in range(_BW // 16):
                        sl = pl.ds(off + k * 16, 16)
                        dsts[p][sl] = dsts[p][sl] * _SCALE + pev
                    out_desc(t, d, p).start()
                    return 0

                lax.fori_loop(0, _TB, tbody, 0)
            return 0

        lax.fori_loop(0, _NBLK // 2, blockpair, 0)

        plsc.subcore_barrier()

        @pl.when(i + 1 < _D_PER_CORE)
        def _():
            @pl.when(s == 0)
            def _():
                tab_stage_desc(d + 1).start()
                tab_stage_desc(d + 1).wait()

            pe_stage_desc(d + 1).start()
            pe_stage_desc(d + 1).wait()

        plsc.subcore_barrier()
        return 0

    lax.fori_loop(0, _D_PER_CORE, dbody, 0)

    dlast = dbase + _D_PER_CORE - 1
    for p in range(2):
        t0 = _T - 2 * _TB + p * _TB

        def odrain_last(j, _, t0=t0, p=p):
            out_desc(t0 + j, dlast, p).wait()
            return 0

        lax.fori_loop(0, _TB, odrain_last, 0)


@jax.jit
def _run(xT, tabT, peS):
    mesh = plsc.VectorSubcoreMesh(core_axis_name="c", subcore_axis_name="s")
    k = functools.partial(
        pl.kernel,
        mesh=mesh,
        out_type=jax.ShapeDtypeStruct((_T, _D, _B), jnp.float32),
        scratch_types=[
            pltpu.VMEM((_T * _BW,), jnp.int32),
            pltpu.VMEM((_TB * _BW,), jnp.float32),
            pltpu.VMEM((_TB * _BW,), jnp.float32),
            pltpu.VMEM((_T * 16,), jnp.float32),
            pltpu.SemaphoreType.DMA,
            pltpu.SemaphoreType.DMA,
            pltpu.SemaphoreType.DMA,
            pltpu.SemaphoreType.DMA,
            pltpu.SemaphoreType.DMA,
            pltpu.VMEM_SHARED((_VOCAB,), jnp.float32),
        ],
    )(_sc_body)
    return k(xT, tabT, peS)


def kernel(x, table):
    peS = _make_pe_splat()
    outT = _run(x.T, table.T, peS)
    return outT.transpose(2, 0, 1)
